# Initial kernel scaffold; baseline (speedup 1.0000x reference)
import jax
import jax.numpy as jnp
from jax.experimental import pallas as pl


def kernel(lap_indices, lap_values, x, unif_noise, weight, p_logit):
    raise NotImplementedError("write your pallas kernel here")


# speedup vs baseline: 11.8224x; 11.8224x over previous
import functools

import jax
import jax.numpy as jnp
from jax import lax
from jax.experimental import pallas as pl
from jax.experimental.pallas import tpu as pltpu
from jax.experimental.pallas import tpu_sc as plsc

B, V, E, FIN, FOUT, K = 8, 10000, 320000, 128, 128, 5

NC = 2
NT = 16
C = 128
NCH = 158
EPT = NCH * C
EP = NT * EPT
RPT = 640
ACC_V = NT * RPT
ZR = 128
BPC = B // NC


def _spmv_chain_body(xf, colsh, rowsh, valsh, m1, m2, m3, m4,
                     cbuf, rbuf, vbuf, gbuf, acc):
    c = lax.axis_index("c")
    s = lax.axis_index("s")

    zero16 = jnp.zeros((16,), jnp.float32)

    outs = [m1, m2, m3, m4]
    for bi in range(BPC):
        b = c * BPC + bi
        for k in range(1, K):
            src = xf if k == 1 else outs[k - 2]

            def zrow(i, carry):
                for q in range(FIN // 16):
                    gbuf[i, pl.ds(q * 16, 16)] = zero16
                return carry

            lax.fori_loop(0, ZR, zrow, 0)
            for z in range(RPT // ZR):
                pltpu.sync_copy(gbuf, acc.at[pl.ds(s * RPT + z * ZR, ZR)])
            plsc.subcore_barrier()

            def chunk(j, carry):
                pltpu.sync_copy(colsh.at[s, j], cbuf)
                pltpu.sync_copy(rowsh.at[s, j], rbuf)
                pltpu.sync_copy(valsh.at[s, j], vbuf)
                boff = b * V
                for q in range(C // 16):
                    sl = pl.ds(q * 16, 16)
                    cbuf[0, sl] = cbuf[0, sl] + boff
                pltpu.sync_copy(src.at[cbuf.at[0]], gbuf)

                def edge16(g, carry2):
                    vv = vbuf[0, pl.ds(g * 16, 16)]
                    for i in range(16):
                        val = vv[i]
                        e = g * 16 + i
                        for q in range(FIN // 16):
                            sl = pl.ds(q * 16, 16)
                            gbuf[e, sl] = gbuf[e, sl] * val
                    return carry2

                lax.fori_loop(0, C // 16, edge16, 0)
                pltpu.sync_copy(gbuf, acc.at[rbuf.at[0]], add=True)
                return carry

            lax.fori_loop(0, NCH, chunk, 0)
            plsc.subcore_barrier()

            dst = outs[k - 1]
            last = V - (NT - 1) * RPT

            @pl.when(s < NT - 1)
            def _drain_full():
                pltpu.sync_copy(acc.at[pl.ds(s * RPT, RPT)],
                                dst.at[pl.ds(b * V + s * RPT, RPT)])

            @pl.when(s == NT - 1)
            def _drain_last():
                pltpu.sync_copy(acc.at[pl.ds((NT - 1) * RPT, last)],
                                dst.at[pl.ds(b * V + (NT - 1) * RPT, last)])

            plsc.subcore_barrier()


_spmv_chain = functools.partial(
    pl.kernel,
    out_type=[jax.ShapeDtypeStruct((B * V, FIN), jnp.float32)] * 4,
    mesh=plsc.VectorSubcoreMesh(core_axis_name="c", subcore_axis_name="s"),
    scratch_types=[
        pltpu.VMEM((1, C), jnp.int32),
        pltpu.VMEM((1, C), jnp.int32),
        pltpu.VMEM((1, C), jnp.float32),
        pltpu.VMEM((C, FIN), jnp.float32),
        pltpu.VMEM_SHARED((ACC_V, FIN), jnp.float32),
    ],
)(_spmv_chain_body)


RB = 2000


def _mm_body(noise_ref, plog_ref, w_ref, x_ref, m1_ref, m2_ref, m3_ref,
             m4_ref, o_ref):
    eps = 1e-7
    temp = 2.0 / 3.0
    p = jax.nn.sigmoid(plog_ref[0])
    u = noise_ref[...].reshape(1, FIN)
    z = (jnp.log(p + eps) - jnp.log(1.0 - p + eps)
         + jnp.log(u + eps) - jnp.log(1.0 - u + eps)) / temp
    scale = (1.0 - jax.nn.sigmoid(z)) / (1.0 - p)

    w0, w1, w2, w3, w4 = (w_ref[i] for i in range(K))
    dot = functools.partial(jnp.dot, preferred_element_type=jnp.float32)
    acc = dot(x_ref[...] * scale, w0 - w2 + w4)
    acc += dot(m1_ref[...] * scale, w1 - 3.0 * w3)
    acc += dot(m2_ref[...] * scale, 2.0 * w2 - 8.0 * w4)
    acc += dot(m3_ref[...] * scale, 4.0 * w3)
    acc += dot(m4_ref[...] * scale, 8.0 * w4)
    o_ref[...] = acc


def _matmul(noise, plog, wk, xf, m1, m2, m3, m4):
    nblk = V // RB
    row_spec = pl.BlockSpec((RB, FIN), lambda b, i: (b * nblk + i, 0))
    return pl.pallas_call(
        _mm_body,
        grid=(B, nblk),
        in_specs=[
            pl.BlockSpec((1, 1, FIN), lambda b, i: (b, 0, 0)),
            pl.BlockSpec(memory_space=pltpu.SMEM),
            pl.BlockSpec((K, FIN, FOUT), lambda b, i: (0, 0, 0)),
            row_spec, row_spec, row_spec, row_spec, row_spec,
        ],
        out_specs=pl.BlockSpec((RB, FOUT), lambda b, i: (b * nblk + i, 0)),
        out_shape=jax.ShapeDtypeStruct((B * V, FOUT), jnp.float32),
    )(noise, plog, wk, xf, m1, m2, m3, m4)


def kernel(lap_indices, lap_values, x, unif_noise, weight, p_logit):
    rows = lap_indices[0].astype(jnp.int32)
    cols = lap_indices[1].astype(jnp.int32)
    vals = lap_values.astype(jnp.float32)
    pad = EP - E
    colsh = jnp.pad(cols, (0, pad)).reshape(NT, NCH, 1, C)
    rowsh = jnp.pad(rows, (0, pad)).reshape(NT, NCH, 1, C)
    valsh = jnp.pad(vals, (0, pad)).reshape(NT, NCH, 1, C)

    xf = x.reshape(B * V, FIN)
    m1, m2, m3, m4 = _spmv_chain(xf, colsh, rowsh, valsh)

    wk = jnp.transpose(weight.reshape(FIN, K, FOUT), (1, 0, 2))
    noise = unif_noise.reshape(B, 1, FIN)
    plog = jnp.reshape(p_logit, (1,)).astype(jnp.float32)
    out = _matmul(noise, plog, wk, xf, m1, m2, m3, m4)
    return out.reshape(B, V, FOUT)


# speedup vs baseline: 28.2810x; 2.3922x over previous
import functools

import jax
import jax.numpy as jnp
from jax import lax
from jax.experimental import pallas as pl
from jax.experimental.pallas import tpu as pltpu
from jax.experimental.pallas import tpu_sc as plsc

B, V, E, FIN, FOUT, K = 8, 10000, 320000, 128, 128, 5

NC = 2
NT = 16
C = 112
NCH = 180
EPT = NCH * C
EP = NT * EPT
RPT = 640
ACC_V = NT * RPT
BPC = B // NC
NQ = FIN // 16
NG = C // 16


def _spmv_chain_body(xf, meta, mval, m1, m2, m3, m4,
                     g0, g1, g2, mb0, mb1, mb2, vb0, vb1, vb2,
                     rb0, rb1, rb2, acc,
                     sg0, sg1, sg2, ss0, ss1, ss2, sm0, sm1, sm2):
    c = lax.axis_index("c")
    s = lax.axis_index("s")
    gb = [g0, g1, g2]
    mb = [mb0, mb1, mb2]
    vb = [vb0, vb1, vb2]
    rb = [rb0, rb1, rb2]
    sg = [sg0, sg1, sg2]
    ss = [ss0, ss1, ss2]
    sm = [sm0, sm1, sm2]
    zero16 = jnp.zeros((16,), jnp.float32)
    outs = [m1, m2, m3, m4]

    def addoff(u, boff):
        for q in range(NG):
            sl = pl.ds(q * 16, 16)
            mb[u][0, sl] = mb[u][0, sl] + boff

    def scale_and_copy_rows(u):
        def grp(g, carry):
            sl16 = pl.ds(g * 16, 16)
            rb[u][0, sl16] = mb[u][1, sl16]
            vv = vb[u][0, sl16]
            for i in range(16):
                val = vv[i]
                e = g * 16 + i
                for q in range(NQ):
                    sl = pl.ds(q * 16, 16)
                    gb[u][e, sl] = gb[u][e, sl] * val
            return carry

        lax.fori_loop(0, NG, grp, 0)

    def run_pass(src, dst, b):
        def zrow(i, carry):
            for q in range(NQ):
                g0[i, pl.ds(q * 16, 16)] = zero16
            return carry

        lax.fori_loop(0, C, zrow, 0)
        for z in range(RPT // C):
            pltpu.sync_copy(g0, acc.at[pl.ds(s * RPT + z * C, C)])
        rem = RPT - (RPT // C) * C
        pltpu.sync_copy(g0.at[pl.ds(0, rem)],
                        acc.at[pl.ds(s * RPT + (RPT // C) * C, rem)])
        plsc.subcore_barrier()

        boff = b * V
        for u in range(3):
            pltpu.async_copy(meta.at[s, u], mb[u], sm[u])
            pltpu.async_copy(mval.at[s, u], vb[u], sm[u])
        for u in range(2):
            pltpu.make_async_copy(meta.at[s, u], mb[u], sm[u]).wait()
            pltpu.make_async_copy(mval.at[s, u], vb[u], sm[u]).wait()
            addoff(u, boff)
            pltpu.async_copy(src.at[mb[u].at[0]], gb[u], sg[u])

        def iter3(jj, carry):
            for u in range(3):
                j = jj * 3 + u
                u2 = (u + 2) % 3
                pltpu.make_async_copy(src.at[mb[u].at[0]], gb[u],
                                      sg[u]).wait()
                scale_and_copy_rows(u)
                pltpu.async_copy(gb[u], acc.at[rb[u].at[0]], ss[u], add=True)

                @pl.when(j + 3 < NCH)
                def _mload():
                    pltpu.async_copy(meta.at[s, j + 3], mb[u], sm[u])
                    pltpu.async_copy(mval.at[s, j + 3], vb[u], sm[u])

                @pl.when(j + 2 < NCH)
                def _prep():
                    pltpu.make_async_copy(meta.at[s, j + 2], mb[u2],
                                          sm[u2]).wait()
                    pltpu.make_async_copy(mval.at[s, j + 2], vb[u2],
                                          sm[u2]).wait()
                    addoff(u2, boff)

                @pl.when(jnp.logical_and(j >= 1, j + 2 < NCH))
                def _free():
                    pltpu.make_async_copy(gb[u2], acc.at[rb[u2].at[0]],
                                          ss[u2]).wait()

                @pl.when(j + 2 < NCH)
                def _gissue():
                    pltpu.async_copy(src.at[mb[u2].at[0]], gb[u2], sg[u2])

            return carry

        lax.fori_loop(0, NCH // 3, iter3, 0)
        for u in range(3):
            pltpu.make_async_copy(gb[u], acc.at[rb[u].at[0]], ss[u]).wait()
        plsc.subcore_barrier()

        last = V - (NT - 1) * RPT

        @pl.when(s < NT - 1)
        def _drain_full():
            pltpu.sync_copy(acc.at[pl.ds(s * RPT, RPT)],
                            dst.at[pl.ds(b * V + s * RPT, RPT)])

        @pl.when(s == NT - 1)
        def _drain_last():
            pltpu.sync_copy(acc.at[pl.ds((NT - 1) * RPT, last)],
                            dst.at[pl.ds(b * V + (NT - 1) * RPT, last)])

        plsc.subcore_barrier()

    def bi_body(bi, carry):
        b = c * BPC + bi
        for k in range(1, K):
            src = xf if k == 1 else outs[k - 2]
            run_pass(src, outs[k - 1], b)
        return carry

    lax.fori_loop(0, BPC, bi_body, 0)


_spmv_chain = functools.partial(
    pl.kernel,
    out_type=[jax.ShapeDtypeStruct((B * V, FIN), jnp.float32)] * 4,
    mesh=plsc.VectorSubcoreMesh(core_axis_name="c", subcore_axis_name="s"),
    scratch_types=[
        pltpu.VMEM((C, FIN), jnp.float32),
        pltpu.VMEM((C, FIN), jnp.float32),
        pltpu.VMEM((C, FIN), jnp.float32),
        pltpu.VMEM((2, C), jnp.int32),
        pltpu.VMEM((2, C), jnp.int32),
        pltpu.VMEM((2, C), jnp.int32),
        pltpu.VMEM((1, C), jnp.float32),
        pltpu.VMEM((1, C), jnp.float32),
        pltpu.VMEM((1, C), jnp.float32),
        pltpu.VMEM((1, C), jnp.int32),
        pltpu.VMEM((1, C), jnp.int32),
        pltpu.VMEM((1, C), jnp.int32),
        pltpu.VMEM_SHARED((ACC_V, FIN), jnp.float32),
        pltpu.SemaphoreType.DMA,
        pltpu.SemaphoreType.DMA,
        pltpu.SemaphoreType.DMA,
        pltpu.SemaphoreType.DMA,
        pltpu.SemaphoreType.DMA,
        pltpu.SemaphoreType.DMA,
        pltpu.SemaphoreType.DMA,
        pltpu.SemaphoreType.DMA,
        pltpu.SemaphoreType.DMA,
    ],
)(_spmv_chain_body)


RB = 2000


def _mm_body(noise_ref, plog_ref, w_ref, x_ref, m1_ref, m2_ref, m3_ref,
             m4_ref, o_ref):
    eps = 1e-7
    temp = 2.0 / 3.0
    p = jax.nn.sigmoid(plog_ref[0])
    u = noise_ref[...].reshape(1, FIN)
    z = (jnp.log(p + eps) - jnp.log(1.0 - p + eps)
         + jnp.log(u + eps) - jnp.log(1.0 - u + eps)) / temp
    scale = (1.0 - jax.nn.sigmoid(z)) / (1.0 - p)

    w0, w1, w2, w3, w4 = (w_ref[i] for i in range(K))
    dot = functools.partial(jnp.dot, preferred_element_type=jnp.float32)
    acc = dot(x_ref[...] * scale, w0 - w2 + w4)
    acc += dot(m1_ref[...] * scale, w1 - 3.0 * w3)
    acc += dot(m2_ref[...] * scale, 2.0 * w2 - 8.0 * w4)
    acc += dot(m3_ref[...] * scale, 4.0 * w3)
    acc += dot(m4_ref[...] * scale, 8.0 * w4)
    o_ref[...] = acc


def _matmul(noise, plog, wk, xf, m1, m2, m3, m4):
    nblk = V // RB
    row_spec = pl.BlockSpec((RB, FIN), lambda b, i: (b * nblk + i, 0))
    return pl.pallas_call(
        _mm_body,
        grid=(B, nblk),
        in_specs=[
            pl.BlockSpec((1, 1, FIN), lambda b, i: (b, 0, 0)),
            pl.BlockSpec(memory_space=pltpu.SMEM),
            pl.BlockSpec((K, FIN, FOUT), lambda b, i: (0, 0, 0)),
            row_spec, row_spec, row_spec, row_spec, row_spec,
        ],
        out_specs=pl.BlockSpec((RB, FOUT), lambda b, i: (b * nblk + i, 0)),
        out_shape=jax.ShapeDtypeStruct((B * V, FOUT), jnp.float32),
    )(noise, plog, wk, xf, m1, m2, m3, m4)


def kernel(lap_indices, lap_values, x, unif_noise, weight, p_logit):
    rows = lap_indices[0].astype(jnp.int32)
    cols = lap_indices[1].astype(jnp.int32)
    pad = EP - E
    meta = jnp.stack([
        jnp.pad(cols, (0, pad)).reshape(NT, NCH, C),
        jnp.pad(rows, (0, pad)).reshape(NT, NCH, C),
    ], axis=2)
    mval = jnp.pad(lap_values.astype(jnp.float32),
                   (0, pad)).reshape(NT, NCH, 1, C)

    xf = x.reshape(B * V, FIN)
    m1, m2, m3, m4 = _spmv_chain(xf, meta, mval)

    wk = jnp.transpose(weight.reshape(FIN, K, FOUT), (1, 0, 2))
    noise = unif_noise.reshape(B, 1, FIN)
    plog = jnp.reshape(p_logit, (1,)).astype(jnp.float32)
    out = _matmul(noise, plog, wk, xf, m1, m2, m3, m4)
    return out.reshape(B, V, FOUT)
